# Initial kernel scaffold; baseline (speedup 1.0000x reference)
import jax
import jax.numpy as jnp
from jax.experimental import pallas as pl


def kernel(bert_inputs, segment_labels, token_table, segment_table):
    raise NotImplementedError("write your pallas kernel here")


# speedup vs baseline: 2.2282x; 2.2282x over previous
import functools
import math

import jax
import jax.numpy as jnp
from jax import lax
from jax.experimental import pallas as pl
from jax.experimental.pallas import tpu as pltpu
from jax.experimental.pallas import tpu_sc as plsc

_D = 64
_L = 200
_B = 4096
_TOTAL = _B * _L
_NC = 2
_NS = 16
_NW = _NC * _NS
_ROWS_PER_W = _TOTAL // _NW
_CHUNK = 512
_STEPS = _ROWS_PER_W // _CHUNK
_GW = 128
_SUB = _CHUNK // _GW
_LANES = 16
_SCALE = 2.0 * math.sqrt(float(_D))


def _make_pe(n, d):
    position = jnp.arange(0, n, dtype=jnp.float32)[:, None]
    div_term = jnp.exp(-jnp.arange(0, d, 2, dtype=jnp.float32) * math.log(10000.0) / d)
    pe = jnp.zeros((n, d), dtype=jnp.float32)
    pe = pe.at[:, 0::2].set(jnp.sin(position * div_term))
    pe = pe.at[:, 1::2].set(jnp.cos(position * div_term))
    return pe


def _sc_lookup(token_table, comb, idx, seg):
    mesh = plsc.VectorSubcoreMesh(core_axis_name="c", subcore_axis_name="s")

    @functools.partial(
        pl.kernel,
        out_type=jax.ShapeDtypeStruct((_TOTAL, _D), jnp.float32),
        mesh=mesh,
        compiler_params=pltpu.CompilerParams(use_tc_tiling_on_sc=False),
        scratch_types=[
            pltpu.VMEM((_CHUNK,), jnp.int32),
            pltpu.VMEM((_CHUNK,), jnp.int32),
            pltpu.VMEM((_CHUNK,), jnp.int32),
            pltpu.VMEM((_CHUNK, _D), jnp.float32),
            pltpu.VMEM((_CHUNK, _D), jnp.float32),
            pltpu.SemaphoreType.DMA,
        ],
    )
    def k(tok_hbm, comb_hbm, idx_hbm, seg_hbm, out_hbm,
          idx_v, seg_v, cidx_v, rows_v, add_v, sem):
        wid = lax.axis_index("s") * _NC + lax.axis_index("c")
        w_base = wid * _ROWS_PER_W

        @pl.loop(0, _STEPS)
        def _step(step):
            base = w_base + step * _CHUNK
            pltpu.sync_copy(idx_hbm.at[pl.ds(base, _CHUNK)], idx_v)
            pltpu.sync_copy(seg_hbm.at[pl.ds(base, _CHUNK)], seg_v)

            @pl.loop(0, _CHUNK, step=_LANES)
            def _cidx(c0):
                pos = base + c0 + lax.broadcasted_iota(jnp.int32, (_LANES,), 0)
                cidx_v[pl.ds(c0, _LANES)] = (
                    seg_v[pl.ds(c0, _LANES)] * _L + lax.rem(pos, _L))

            copies = []
            for j in range(_SUB):
                copies.append(pltpu.async_copy(
                    tok_hbm.at[idx_v.at[pl.ds(j * _GW, _GW)]],
                    rows_v.at[pl.ds(j * _GW, _GW)], sem))
                copies.append(pltpu.async_copy(
                    comb_hbm.at[cidx_v.at[pl.ds(j * _GW, _GW)]],
                    add_v.at[pl.ds(j * _GW, _GW)], sem))
            for c in copies:
                c.wait()

            @pl.loop(0, _CHUNK)
            def _fma(r):
                for c0 in range(0, _D, _LANES):
                    slc = (pl.ds(r, 1), pl.ds(c0, _LANES))
                    rows_v[slc] = rows_v[slc] * _SCALE + add_v[slc]

            pltpu.sync_copy(rows_v, out_hbm.at[pl.ds(base, _CHUNK)])

    return k(token_table, comb, idx, seg)


def kernel(bert_inputs, segment_labels, token_table, segment_table):
    pe = _make_pe(_L, _D)
    comb = (segment_table[:, None, :].astype(jnp.float32)
            + pe[None, :, :]).reshape(3 * _L, _D)
    idx = bert_inputs.reshape(_TOTAL).astype(jnp.int32)
    seg = segment_labels.reshape(_TOTAL).astype(jnp.int32)
    out = _sc_lookup(token_table.astype(jnp.float32), comb, idx, seg)
    return out.reshape(_B, _L, _D)


# speedup vs baseline: 2.2611x; 1.0147x over previous
import functools
import math

import jax
import jax.numpy as jnp
from jax import lax
from jax.experimental import pallas as pl
from jax.experimental.pallas import tpu as pltpu
from jax.experimental.pallas import tpu_sc as plsc

_D = 64
_L = 200
_B = 4096
_TOTAL = _B * _L
_NC = 2
_NS = 16
_NW = _NC * _NS
_ROWS_PER_W = _TOTAL // _NW
_CHUNK = 512
_STEPS = _ROWS_PER_W // _CHUNK
_GW = 128
_SUB = _CHUNK // _GW
_LANES = 16
_SCALE = 2.0 * math.sqrt(float(_D))


def _make_pe(n, d):
    position = jnp.arange(0, n, dtype=jnp.float32)[:, None]
    div_term = jnp.exp(-jnp.arange(0, d, 2, dtype=jnp.float32) * math.log(10000.0) / d)
    pe = jnp.zeros((n, d), dtype=jnp.float32)
    pe = pe.at[:, 0::2].set(jnp.sin(position * div_term))
    pe = pe.at[:, 1::2].set(jnp.cos(position * div_term))
    return pe


def _sc_lookup(token_table, comb, idx, seg):
    mesh = plsc.VectorSubcoreMesh(core_axis_name="c", subcore_axis_name="s")

    @functools.partial(
        pl.kernel,
        out_type=jax.ShapeDtypeStruct((_TOTAL * _D,), jnp.float32),
        mesh=mesh,
        compiler_params=pltpu.CompilerParams(use_tc_tiling_on_sc=False),
        scratch_types=[
            pltpu.VMEM((_CHUNK,), jnp.int32),
            pltpu.VMEM((_CHUNK,), jnp.int32),
            pltpu.VMEM((_CHUNK,), jnp.int32),
            pltpu.VMEM((_CHUNK, _D), jnp.float32),
            pltpu.VMEM((_CHUNK, _D), jnp.float32),
            pltpu.VMEM((_CHUNK * _D,), jnp.float32),
            pltpu.SemaphoreType.DMA,
        ],
    )
    def k(tok_hbm, comb_hbm, idx_hbm, seg_hbm, out_hbm,
          idx_v, seg_v, cidx_v, rows_v, add_v, out_v, sem):
        wid = lax.axis_index("s") * _NC + lax.axis_index("c")
        w_base = wid * _ROWS_PER_W

        @pl.loop(0, _STEPS)
        def _step(step):
            base = w_base + step * _CHUNK
            pltpu.sync_copy(idx_hbm.at[pl.ds(base, _CHUNK)], idx_v)
            pltpu.sync_copy(seg_hbm.at[pl.ds(base, _CHUNK)], seg_v)

            @pl.loop(0, _CHUNK, step=_LANES)
            def _cidx(c0):
                pos = base + c0 + lax.broadcasted_iota(jnp.int32, (_LANES,), 0)
                cidx_v[pl.ds(c0, _LANES)] = (
                    seg_v[pl.ds(c0, _LANES)] * _L + lax.rem(pos, _L))

            copies = []
            for j in range(_SUB):
                copies.append(pltpu.async_copy(
                    tok_hbm.at[idx_v.at[pl.ds(j * _GW, _GW)]],
                    rows_v.at[pl.ds(j * _GW, _GW)], sem))
                copies.append(pltpu.async_copy(
                    comb_hbm.at[cidx_v.at[pl.ds(j * _GW, _GW)]],
                    add_v.at[pl.ds(j * _GW, _GW)], sem))
            for c in copies:
                c.wait()

            @pl.loop(0, _CHUNK)
            def _fma(r):
                for c0 in range(0, _D, _LANES):
                    slc = (pl.ds(r, 1), pl.ds(c0, _LANES))
                    out_v[pl.ds(r * _D + c0, _LANES)] = (
                        rows_v[slc] * _SCALE + add_v[slc]).reshape(_LANES)

            pltpu.sync_copy(out_v, out_hbm.at[pl.ds(base * _D, _CHUNK * _D)])

    return k(token_table, comb, idx, seg)


def kernel(bert_inputs, segment_labels, token_table, segment_table):
    pe = _make_pe(_L, _D)
    comb = (segment_table[:, None, :].astype(jnp.float32)
            + pe[None, :, :]).reshape(3 * _L, _D)
    idx = bert_inputs.reshape(_TOTAL).astype(jnp.int32)
    seg = segment_labels.reshape(_TOTAL).astype(jnp.int32)
    out = _sc_lookup(token_table.astype(jnp.float32), comb, idx, seg)
    return out.reshape(_B, _L, _D)
